# Initial kernel scaffold; baseline (speedup 1.0000x reference)
import jax
import jax.numpy as jnp
from jax.experimental import pallas as pl


def kernel(x, dis, spec, edge_src, params):
    raise NotImplementedError("write your pallas kernel here")


# speedup vs baseline: 1.4420x; 1.4420x over previous
import functools

import jax
import jax.numpy as jnp
from jax import lax
from jax.experimental import pallas as pl
from jax.experimental.pallas import tpu as pltpu
from jax.experimental.pallas import tpu_sc as plsc

N = 10000
K = 32
CN = 6
E = N * K

_LEAK = 0.01


def _leaky(x):
    return jnp.where(x >= 0, x, _LEAK * x)


def _sc_gather(table, idx_3d, D, n_chunks, nbuf):
    info = plsc.get_sparse_core_info()
    nc, ns = info.num_cores, info.num_subcores
    nw = nc * ns
    per_w = n_chunks * 128
    S = nbuf * 128
    n_super = n_chunks // nbuf
    mesh = plsc.VectorSubcoreMesh(core_axis_name="c", subcore_axis_name="s")

    @functools.partial(
        pl.kernel,
        mesh=mesh,
        compiler_params=pltpu.CompilerParams(use_tc_tiling_on_sc=False),
        out_type=jax.ShapeDtypeStruct((nw * per_w, D), jnp.float32),
        scratch_types=[
            pltpu.VMEM((n_chunks, 128), jnp.int32),
            pltpu.VMEM((S, D), jnp.float32),
            pltpu.SemaphoreType.DMA,
        ],
    )
    def k(table_hbm, idx_hbm, out_hbm, idx_v, buf, sem):
        wid = lax.axis_index("s") * nc + lax.axis_index("c")
        base = wid * per_w
        pltpu.sync_copy(idx_hbm.at[wid], idx_v)

        def body(sg, carry):
            cps = []
            for b in range(nbuf):
                cps.append(pltpu.async_copy(
                    table_hbm.at[idx_v.at[sg * nbuf + b]],
                    buf.at[pl.ds(b * 128, 128)], sem))
            for cp in cps:
                cp.wait()
            pltpu.sync_copy(buf, out_hbm.at[pl.ds(base + sg * S, S)])
            return carry

        lax.fori_loop(0, n_super, body, 0)

    return k(table, idx_3d)


def _prep_kernel(x_ref, dis_ref, spec_ref, hfcW_ref, hfcb_ref, attnW_ref,
                 wfcW_ref, wfcb_ref, wfc1W_ref, wfc1b_ref,
                 feat_ref, ew_ref):
    xv = x_ref[...]
    z = jnp.dot(xv, hfcW_ref[...].T,
                preferred_element_type=jnp.float32) + hfcb_ref[...]
    w1 = attnW_ref[:, 0:CN]
    s = jnp.sum(attnW_ref[:, CN:2 * CN])
    c = jnp.sum(z * w1, axis=1, keepdims=True)
    a = _leaky(c + s * z)
    m = jnp.max(a, axis=1, keepdims=True)
    e = jnp.exp(a - m)
    alpha = e / jnp.sum(e, axis=1, keepdims=True)
    feat = alpha * z
    feat_ref[...] = jnp.concatenate(
        [feat, jnp.zeros((feat.shape[0], 16 - CN), jnp.float32)], axis=1)

    A = jnp.dot(wfc1W_ref[...], wfcW_ref[...],
                preferred_element_type=jnp.float32)
    c0 = jnp.dot(wfc1W_ref[...], wfcb_ref[...],
                 preferred_element_type=jnp.float32) + wfc1b_ref[...]
    ew_ref[...] = (dis_ref[...] * A[0:1, 0:1] + spec_ref[...] * A[0:1, 1:2]
                   + c0[0:1, 0:1])


def _lstm_step(msg_ref, ew_ref, h_s, c_s, wih, whh, bias, B, D, H):
    t = pl.program_id(1)

    @pl.when(t == 0)
    def _():
        h_s[...] = jnp.zeros((B, H), jnp.float32)
        c_s[...] = jnp.zeros((B, H), jnp.float32)

    h = h_s[...]
    c = c_s[...]
    onehot = (lax.broadcasted_iota(jnp.int32, (K, H), 0) == t
              ).astype(jnp.float32)
    ewc = jnp.dot(ew_ref[...], onehot, preferred_element_type=jnp.float32)
    xt = msg_ref[0]
    gi_ = jnp.dot(xt, wih[0][...], preferred_element_type=jnp.float32) * ewc \
        + jnp.dot(h, whh[0][...], preferred_element_type=jnp.float32) + bias[0][...]
    gf_ = jnp.dot(xt, wih[1][...], preferred_element_type=jnp.float32) * ewc \
        + jnp.dot(h, whh[1][...], preferred_element_type=jnp.float32) + bias[1][...]
    gg_ = jnp.dot(xt, wih[2][...], preferred_element_type=jnp.float32) * ewc \
        + jnp.dot(h, whh[2][...], preferred_element_type=jnp.float32) + bias[2][...]
    go_ = jnp.dot(xt, wih[3][...], preferred_element_type=jnp.float32) * ewc \
        + jnp.dot(h, whh[3][...], preferred_element_type=jnp.float32) + bias[3][...]
    i = jax.nn.sigmoid(gi_)
    f = jax.nn.sigmoid(gf_)
    gg = jnp.tanh(gg_)
    o = jax.nn.sigmoid(go_)
    c2 = f * c + i * gg
    h2 = o * jnp.tanh(c2)
    h_s[...] = h2
    c_s[...] = c2
    return h2


def _lstm1_kernel(msg_ref, ew_ref, feat_ref,
                  wih0, wih1, wih2, wih3, whh0, whh1, whh2, whh3,
                  b0, b1, b2, b3, Ws_ref, Wn_ref, cb_ref, h1_ref,
                  h_s, c_s, *, B):
    h = _lstm_step(msg_ref, ew_ref, h_s, c_s,
                   (wih0, wih1, wih2, wih3), (whh0, whh1, whh2, whh3),
                   (b0, b1, b2, b3), B, 16, CN)

    @pl.when(pl.program_id(1) == K - 1)
    def _():
        out = (jnp.dot(feat_ref[...], Ws_ref[...], preferred_element_type=jnp.float32)
               + jnp.dot(h, Wn_ref[...], preferred_element_type=jnp.float32)
               + cb_ref[...])
        h1_ref[...] = _leaky(out)


def _lstm2_kernel(msg_ref, ew_ref, h1_ref,
                  wih0, wih1, wih2, wih3, whh0, whh1, whh2, whh3,
                  b0, b1, b2, b3, Ws_ref, Wn_ref, cb_ref,
                  bng_ref, bnb_ref, bnrm_ref, bnrv_ref, h2_ref,
                  h_s, c_s, *, B):
    h = _lstm_step(msg_ref, ew_ref, h_s, c_s,
                   (wih0, wih1, wih2, wih3), (whh0, whh1, whh2, whh3),
                   (b0, b1, b2, b3), B, 32, 32)

    @pl.when(pl.program_id(1) == K - 1)
    def _():
        out = (jnp.dot(h1_ref[...], Ws_ref[...], preferred_element_type=jnp.float32)
               + jnp.dot(h, Wn_ref[...], preferred_element_type=jnp.float32)
               + cb_ref[...])
        scale = bng_ref[...] * lax.rsqrt(bnrv_ref[...] + 1e-5)
        shift = bnb_ref[...] - bnrm_ref[...] * scale
        h2_ref[...] = _leaky(out * scale + shift)


def _head_kernel(h2_ref, gWrep_ref, f1W_ref, f1b_ref, f2W_ref, f2b_ref,
                 out_ref):
    h2 = h2_ref[...]
    l = jnp.dot(h2, gWrep_ref[...], preferred_element_type=jnp.float32)
    m = jnp.max(l)
    e = jnp.exp(l - m)
    s = jnp.sum(e, axis=0, keepdims=True)
    gate = e / s
    pooled = jnp.sum(gate[:, 0:20] * h2, axis=0, keepdims=True)
    o1 = _leaky(jnp.dot(pooled, f1W_ref[...].T,
                        preferred_element_type=jnp.float32) + f1b_ref[...])
    out_ref[...] = jnp.dot(o1, f2W_ref[...].T,
                           preferred_element_type=jnp.float32) + f2b_ref[...]


def _split4_T(W, b2, H):
    ws = tuple(W[gi * H:(gi + 1) * H, :].T for gi in range(4))
    bs = tuple(b2[:, gi * H:(gi + 1) * H] for gi in range(4))
    return ws, bs


def kernel(x, dis, spec, edge_src, params):
    p = params
    x2d = x.reshape(N, CN)
    dis2d = dis.reshape(E // 128, 128)
    spec2d = spec.reshape(E // 128, 128)

    feat_pad, ew2d = pl.pallas_call(
        _prep_kernel,
        out_shape=(jax.ShapeDtypeStruct((N, 16), jnp.float32),
                   jax.ShapeDtypeStruct((E // 128, 128), jnp.float32)),
    )(x2d, dis2d, spec2d,
      p['hfc_W'], p['hfc_b'].reshape(1, CN), p['attn_W'],
      p['wfc_W'], p['wfc_b'].reshape(100, 1), p['wfc1_W'],
      p['wfc1_b'].reshape(1, 1))

    ew_nk = ew2d.reshape(N, K)

    info = plsc.get_sparse_core_info()
    nw = info.num_cores * info.num_subcores
    n_chunks = -(-E // (nw * 128))
    e_pad = nw * n_chunks * 128
    idx_t = edge_src.reshape(N, K).T.reshape(E)
    idx_pad = jnp.concatenate(
        [idx_t, jnp.zeros((e_pad - E,), jnp.int32)]).reshape(nw, n_chunks, 128)

    msg1 = _sc_gather(feat_pad, idx_pad, 16, n_chunks, nbuf=8)
    msg1 = msg1[:E].reshape(K, N, 16)

    B1 = 5000
    g1 = N // B1
    (wih1, bih1) = _split4_T(p['l1_Wih'], (p['l1_bih'] + p['l1_bhh']).reshape(1, 24), CN)
    wih1 = tuple(jnp.pad(w, ((0, 16 - CN), (0, 0))) for w in wih1)
    (whh1, _) = _split4_T(p['l1_Whh'], jnp.zeros((1, 24)), CN)
    ws1_pad = jnp.pad(p['c1_Ws'].T, ((0, 16 - CN), (0, 0)))

    def full(shape):
        return pl.BlockSpec(shape, lambda i, t: tuple(0 for _ in shape))

    h1 = pl.pallas_call(
        functools.partial(_lstm1_kernel, B=B1),
        grid=(g1, K),
        in_specs=[
            pl.BlockSpec((1, B1, 16), lambda i, t: (t, i, 0)),
            pl.BlockSpec((B1, K), lambda i, t: (i, 0)),
            pl.BlockSpec((B1, 16), lambda i, t: (i, 0)),
            *[full(w.shape) for w in wih1],
            *[full(w.shape) for w in whh1],
            *[full(b.shape) for b in bih1],
            full(ws1_pad.shape), full((CN, K)), full((1, 32)),
        ],
        out_specs=pl.BlockSpec((B1, K), lambda i, t: (i, 0)),
        out_shape=jax.ShapeDtypeStruct((N, K), jnp.float32),
        scratch_shapes=[pltpu.VMEM((B1, CN), jnp.float32),
                        pltpu.VMEM((B1, CN), jnp.float32)],
    )(msg1, ew_nk, feat_pad, *wih1, *whh1, *bih1,
      ws1_pad, p['c1_Wn'].T, p['c1_b'].reshape(1, 32))

    msg2 = _sc_gather(h1, idx_pad, 32, n_chunks, nbuf=8)
    msg2 = msg2[:E].reshape(K, N, 32)

    B2 = 5000
    g2 = N // B2
    (wih2, bih2) = _split4_T(p['l2_Wih'], (p['l2_bih'] + p['l2_bhh']).reshape(1, 128), 32)
    (whh2, _) = _split4_T(p['l2_Whh'], jnp.zeros((1, 128)), 32)

    h2 = pl.pallas_call(
        functools.partial(_lstm2_kernel, B=B2),
        grid=(g2, K),
        in_specs=[
            pl.BlockSpec((1, B2, 32), lambda i, t: (t, i, 0)),
            pl.BlockSpec((B2, K), lambda i, t: (i, 0)),
            pl.BlockSpec((B2, K), lambda i, t: (i, 0)),
            *[full(w.shape) for w in wih2],
            *[full(w.shape) for w in whh2],
            *[full(b.shape) for b in bih2],
            full((32, 20)), full((32, 20)), full((1, 20)),
            full((1, 20)), full((1, 20)), full((1, 20)), full((1, 20)),
        ],
        out_specs=pl.BlockSpec((B2, 20), lambda i, t: (i, 0)),
        out_shape=jax.ShapeDtypeStruct((N, 20), jnp.float32),
        scratch_shapes=[pltpu.VMEM((B2, 32), jnp.float32),
                        pltpu.VMEM((B2, 32), jnp.float32)],
    )(msg2, ew_nk, h1, *wih2, *whh2, *bih2,
      p['c2_Ws'].T, p['c2_Wn'].T, p['c2_b'].reshape(1, 20),
      p['bn_g'].reshape(1, 20), p['bn_b'].reshape(1, 20),
      p['bn_rm'].reshape(1, 20), p['bn_rv'].reshape(1, 20))

    out = pl.pallas_call(
        _head_kernel,
        out_shape=jax.ShapeDtypeStruct((1, 2), jnp.float32),
    )(h2, jnp.tile(p['gate_W'].T, (1, 128)),
      p['fc1_W'], p['fc1_b'].reshape(1, 10),
      p['fc2_W'], p['fc2_b'].reshape(1, 2))

    return out


# speedup vs baseline: 1.7186x; 1.1918x over previous
import functools

import jax
import jax.numpy as jnp
from jax import lax
from jax.experimental import pallas as pl
from jax.experimental.pallas import tpu as pltpu
from jax.experimental.pallas import tpu_sc as plsc

N = 10000
K = 32
CN = 6
E = N * K

_LEAK = 0.01


def _leaky(x):
    return jnp.where(x >= 0, x, _LEAK * x)


def _sc_gather(table, idx_3d, D, n_chunks, nbuf):
    info = plsc.get_sparse_core_info()
    nc, ns = info.num_cores, info.num_subcores
    nw = nc * ns
    per_w = n_chunks * 128
    S = nbuf * 128
    n_super = n_chunks // nbuf
    mesh = plsc.VectorSubcoreMesh(core_axis_name="c", subcore_axis_name="s")

    @functools.partial(
        pl.kernel,
        mesh=mesh,
        compiler_params=pltpu.CompilerParams(use_tc_tiling_on_sc=False),
        out_type=jax.ShapeDtypeStruct((nw * per_w, D), jnp.float32),
        scratch_types=[
            pltpu.VMEM((n_chunks, 128), jnp.int32),
            pltpu.VMEM((S, D), jnp.float32),
            pltpu.SemaphoreType.DMA,
        ],
    )
    def k(table_hbm, idx_hbm, out_hbm, idx_v, buf, sem):
        wid = lax.axis_index("s") * nc + lax.axis_index("c")
        base = wid * per_w
        pltpu.sync_copy(idx_hbm.at[wid], idx_v)

        def body(sg, carry):
            cps = []
            for b in range(nbuf):
                cps.append(pltpu.async_copy(
                    table_hbm.at[idx_v.at[sg * nbuf + b]],
                    buf.at[pl.ds(b * 128, 128)], sem))
            for cp in cps:
                cp.wait()
            pltpu.sync_copy(buf, out_hbm.at[pl.ds(base + sg * S, S)])
            return carry

        lax.fori_loop(0, n_super, body, 0)

    return k(table, idx_3d)


def _prep_kernel(x_ref, dis_ref, spec_ref, hfcW_ref, hfcb_ref, attnW_ref,
                 wfcW_ref, wfcb_ref, wfc1W_ref, wfc1b_ref,
                 feat_ref, ew_ref):
    xv = x_ref[...]
    z = jnp.dot(xv, hfcW_ref[...].T,
                preferred_element_type=jnp.float32) + hfcb_ref[...]
    w1 = attnW_ref[:, 0:CN]
    s = jnp.sum(attnW_ref[:, CN:2 * CN])
    c = jnp.sum(z * w1, axis=1, keepdims=True)
    a = _leaky(c + s * z)
    m = jnp.max(a, axis=1, keepdims=True)
    e = jnp.exp(a - m)
    alpha = e / jnp.sum(e, axis=1, keepdims=True)
    feat = alpha * z
    feat_ref[...] = jnp.concatenate(
        [feat, jnp.zeros((feat.shape[0], 16 - CN), jnp.float32)], axis=1)

    A = jnp.dot(wfc1W_ref[...], wfcW_ref[...],
                preferred_element_type=jnp.float32)
    c0 = jnp.dot(wfc1W_ref[...], wfcb_ref[...],
                 preferred_element_type=jnp.float32) + wfc1b_ref[...]
    ew_ref[...] = (dis_ref[...] * A[0:1, 0:1] + spec_ref[...] * A[0:1, 1:2]
                   + c0[0:1, 0:1])


def _lstm_step(msg_ref, ew_ref, h_s, c_s, wihA_ref, whhA_ref, bA_ref, B):
    t = pl.program_id(1)

    @pl.when(t == 0)
    def _():
        h_s[...] = jnp.zeros((B, 32), jnp.float32)
        c_s[...] = jnp.zeros((B, 32), jnp.float32)

    h = h_s[...]
    c = c_s[...]
    onehot = (lax.broadcasted_iota(jnp.int32, (K, 128), 0) == t
              ).astype(jnp.float32)
    ewc = jnp.dot(ew_ref[...], onehot, preferred_element_type=jnp.float32)
    xt = msg_ref[0]
    g = (jnp.dot(xt, wihA_ref[...], preferred_element_type=jnp.float32) * ewc
         + jnp.dot(h, whhA_ref[...], preferred_element_type=jnp.float32)
         + bA_ref[...])
    sg = jax.nn.sigmoid(g)
    th = jnp.tanh(g)
    lane = lax.broadcasted_iota(jnp.int32, (B, 128), 1)
    act = jnp.where((lane >= 64) & (lane < 96), th, sg)
    i = act[:, 0:32]
    f = act[:, 32:64]
    gg = act[:, 64:96]
    o = act[:, 96:128]
    c2 = f * c + i * gg
    h2 = o * jnp.tanh(c2)
    h_s[...] = h2
    c_s[...] = c2
    return h2


def _lstm1_kernel(msg_ref, ew_ref, feat_ref, wihA_ref, whhA_ref, bA_ref,
                  Ws_ref, Wn_ref, cb_ref, h1_ref, h_s, c_s, *, B):
    h = _lstm_step(msg_ref, ew_ref, h_s, c_s, wihA_ref, whhA_ref, bA_ref, B)

    @pl.when(pl.program_id(1) == K - 1)
    def _():
        out = (jnp.dot(feat_ref[...], Ws_ref[...], preferred_element_type=jnp.float32)
               + jnp.dot(h, Wn_ref[...], preferred_element_type=jnp.float32)
               + cb_ref[...])
        h1_ref[...] = _leaky(out)


def _lstm2_kernel(msg_ref, ew_ref, h1_ref, wihA_ref, whhA_ref, bA_ref,
                  Ws_ref, Wn_ref, cb_ref,
                  bng_ref, bnb_ref, bnrm_ref, bnrv_ref, h2_ref,
                  h_s, c_s, *, B):
    h = _lstm_step(msg_ref, ew_ref, h_s, c_s, wihA_ref, whhA_ref, bA_ref, B)

    @pl.when(pl.program_id(1) == K - 1)
    def _():
        out = (jnp.dot(h1_ref[...], Ws_ref[...], preferred_element_type=jnp.float32)
               + jnp.dot(h, Wn_ref[...], preferred_element_type=jnp.float32)
               + cb_ref[...])
        scale = bng_ref[...] * lax.rsqrt(bnrv_ref[...] + 1e-5)
        shift = bnb_ref[...] - bnrm_ref[...] * scale
        h2_ref[...] = _leaky(out * scale + shift)


def _head_kernel(h2_ref, gWrep_ref, f1W_ref, f1b_ref, f2W_ref, f2b_ref,
                 out_ref):
    h2 = h2_ref[...]
    l = jnp.dot(h2, gWrep_ref[...], preferred_element_type=jnp.float32)
    m = jnp.max(l)
    e = jnp.exp(l - m)
    s = jnp.sum(e, axis=0, keepdims=True)
    gate = e / s
    pooled = jnp.sum(gate[:, 0:20] * h2, axis=0, keepdims=True)
    o1 = _leaky(jnp.dot(pooled, f1W_ref[...].T,
                        preferred_element_type=jnp.float32) + f1b_ref[...])
    out_ref[...] = jnp.dot(o1, f2W_ref[...].T,
                           preferred_element_type=jnp.float32) + f2b_ref[...]


def _pack_lstm(Wih, Whh, bih, bhh, H, D_pad):
    Din = Wih.shape[1]
    wihA = jnp.zeros((D_pad, 128), jnp.float32)
    whhA = jnp.zeros((32, 128), jnp.float32)
    bA = jnp.zeros((1, 128), jnp.float32)
    bsum = bih + bhh
    for gi in range(4):
        wihA = wihA.at[0:Din, gi * 32:gi * 32 + H].set(Wih[gi * H:(gi + 1) * H, :].T)
        whhA = whhA.at[0:H, gi * 32:gi * 32 + H].set(Whh[gi * H:(gi + 1) * H, :].T)
        bA = bA.at[0:1, gi * 32:gi * 32 + H].set(bsum[gi * H:(gi + 1) * H].reshape(1, H))
    return wihA, whhA, bA


def kernel(x, dis, spec, edge_src, params):
    p = params
    x2d = x.reshape(N, CN)
    dis2d = dis.reshape(E // 128, 128)
    spec2d = spec.reshape(E // 128, 128)

    feat_pad, ew2d = pl.pallas_call(
        _prep_kernel,
        out_shape=(jax.ShapeDtypeStruct((N, 16), jnp.float32),
                   jax.ShapeDtypeStruct((E // 128, 128), jnp.float32)),
    )(x2d, dis2d, spec2d,
      p['hfc_W'], p['hfc_b'].reshape(1, CN), p['attn_W'],
      p['wfc_W'], p['wfc_b'].reshape(100, 1), p['wfc1_W'],
      p['wfc1_b'].reshape(1, 1))

    ew_nk = ew2d.reshape(N, K)

    info = plsc.get_sparse_core_info()
    nw = info.num_cores * info.num_subcores
    n_chunks = -(-E // (nw * 128))
    e_pad = nw * n_chunks * 128
    idx_t = edge_src.reshape(N, K).T.reshape(E)
    idx_pad = jnp.concatenate(
        [idx_t, jnp.zeros((e_pad - E,), jnp.int32)]).reshape(nw, n_chunks, 128)

    msg1 = _sc_gather(feat_pad, idx_pad, 16, n_chunks, nbuf=8)
    msg1 = msg1[:E].reshape(K, N, 16)

    B1 = N
    wihA1, whhA1, bA1 = _pack_lstm(p['l1_Wih'], p['l1_Whh'],
                                   p['l1_bih'], p['l1_bhh'], CN, 16)
    ws1_pad = jnp.pad(p['c1_Ws'].T, ((0, 16 - CN), (0, 0)))
    wn1_pad = jnp.pad(p['c1_Wn'].T, ((0, 32 - CN), (0, 0)))

    def full(shape):
        return pl.BlockSpec(shape, lambda i, t: tuple(0 for _ in shape))

    h1 = pl.pallas_call(
        functools.partial(_lstm1_kernel, B=B1),
        grid=(1, K),
        in_specs=[
            pl.BlockSpec((1, B1, 16), lambda i, t: (t, i, 0)),
            pl.BlockSpec((B1, K), lambda i, t: (i, 0)),
            pl.BlockSpec((B1, 16), lambda i, t: (i, 0)),
            full((16, 128)), full((32, 128)), full((1, 128)),
            full((16, 32)), full((32, 32)), full((1, 32)),
        ],
        out_specs=pl.BlockSpec((B1, K), lambda i, t: (i, 0)),
        out_shape=jax.ShapeDtypeStruct((N, K), jnp.float32),
        scratch_shapes=[pltpu.VMEM((B1, 32), jnp.float32),
                        pltpu.VMEM((B1, 32), jnp.float32)],
    )(msg1, ew_nk, feat_pad, wihA1, whhA1, bA1,
      ws1_pad, wn1_pad, p['c1_b'].reshape(1, 32))

    msg2 = _sc_gather(h1, idx_pad, 32, n_chunks, nbuf=8)
    msg2 = msg2[:E].reshape(K, N, 32)

    B2 = N
    wihA2, whhA2, bA2 = _pack_lstm(p['l2_Wih'], p['l2_Whh'],
                                   p['l2_bih'], p['l2_bhh'], 32, 32)

    h2 = pl.pallas_call(
        functools.partial(_lstm2_kernel, B=B2),
        grid=(1, K),
        in_specs=[
            pl.BlockSpec((1, B2, 32), lambda i, t: (t, i, 0)),
            pl.BlockSpec((B2, K), lambda i, t: (i, 0)),
            pl.BlockSpec((B2, K), lambda i, t: (i, 0)),
            full((32, 128)), full((32, 128)), full((1, 128)),
            full((32, 20)), full((32, 20)), full((1, 20)),
            full((1, 20)), full((1, 20)), full((1, 20)), full((1, 20)),
        ],
        out_specs=pl.BlockSpec((B2, 20), lambda i, t: (i, 0)),
        out_shape=jax.ShapeDtypeStruct((N, 20), jnp.float32),
        scratch_shapes=[pltpu.VMEM((B2, 32), jnp.float32),
                        pltpu.VMEM((B2, 32), jnp.float32)],
    )(msg2, ew_nk, h1, wihA2, whhA2, bA2,
      p['c2_Ws'].T, p['c2_Wn'].T, p['c2_b'].reshape(1, 20),
      p['bn_g'].reshape(1, 20), p['bn_b'].reshape(1, 20),
      p['bn_rm'].reshape(1, 20), p['bn_rv'].reshape(1, 20))

    out = pl.pallas_call(
        _head_kernel,
        out_shape=jax.ShapeDtypeStruct((1, 2), jnp.float32),
    )(h2, jnp.tile(p['gate_W'].T, (1, 128)),
      p['fc1_W'], p['fc1_b'].reshape(1, 10),
      p['fc2_W'], p['fc2_b'].reshape(1, 2))

    return out


# speedup vs baseline: 4.6293x; 2.6936x over previous
import functools

import numpy as np
import jax
import jax.numpy as jnp
from jax import lax
from jax.experimental import pallas as pl
from jax.experimental.pallas import tpu as pltpu
from jax.experimental.pallas import tpu_sc as plsc

N = 10000
K = 32
CN = 6
E = N * K

_LEAK = 0.01


def _leaky(x):
    return jnp.where(x >= 0, x, _LEAK * x)


def _sc_gather(table, idx_2d, D):
    info = plsc.get_sparse_core_info()
    nc, ns = info.num_cores, info.num_subcores
    nw = nc * ns
    per_w = idx_2d.shape[1]
    n_full = per_w // 128
    tail = per_w - n_full * 128
    nbuf = 6
    n_super = n_full // nbuf
    S = nbuf * 128
    mesh = plsc.VectorSubcoreMesh(core_axis_name="c", subcore_axis_name="s")

    @functools.partial(
        pl.kernel,
        mesh=mesh,
        compiler_params=pltpu.CompilerParams(use_tc_tiling_on_sc=False),
        out_type=jax.ShapeDtypeStruct((nw * per_w, D), jnp.float32),
        scratch_types=[
            pltpu.VMEM((per_w,), jnp.int32),
            pltpu.VMEM((S, D), jnp.float32),
            pltpu.SemaphoreType.DMA,
        ],
    )
    def k(table_hbm, idx_hbm, out_hbm, idx_v, buf, sem):
        wid = lax.axis_index("s") * nc + lax.axis_index("c")
        base = wid * per_w
        pltpu.sync_copy(idx_hbm.at[wid], idx_v)

        def body(sg, carry):
            cps = []
            for b in range(nbuf):
                cps.append(pltpu.async_copy(
                    table_hbm.at[idx_v.at[pl.ds((sg * nbuf + b) * 128, 128)]],
                    buf.at[pl.ds(b * 128, 128)], sem))
            for cp in cps:
                cp.wait()
            pltpu.sync_copy(buf, out_hbm.at[pl.ds(base + sg * S, S)])
            return carry

        lax.fori_loop(0, n_super, body, 0)
        if tail:
            pltpu.async_copy(
                table_hbm.at[idx_v.at[pl.ds(n_full * 128, tail)]],
                buf.at[pl.ds(0, tail)], sem).wait()
            pltpu.sync_copy(buf.at[pl.ds(0, tail)],
                            out_hbm.at[pl.ds(base + n_full * 128, tail)])

    return k(table, idx_2d)


def _prep_kernel(x_ref, dis_ref, spec_ref, hfcW_ref, hfcb_ref, attnW_ref,
                 wfcW_ref, wfcb_ref, wfc1W_ref, wfc1b_ref,
                 feat_ref, ew_ref):
    xv = x_ref[...]
    z = jnp.dot(xv, hfcW_ref[...].T,
                preferred_element_type=jnp.float32) + hfcb_ref[...]
    w1 = attnW_ref[:, 0:CN]
    s = jnp.sum(attnW_ref[:, CN:2 * CN])
    c = jnp.sum(z * w1, axis=1, keepdims=True)
    a = _leaky(c + s * z)
    m = jnp.max(a, axis=1, keepdims=True)
    e = jnp.exp(a - m)
    alpha = e / jnp.sum(e, axis=1, keepdims=True)
    feat = alpha * z
    feat_ref[...] = jnp.concatenate(
        [feat, jnp.zeros((feat.shape[0], 16 - CN), jnp.float32)], axis=1)

    A = jnp.dot(wfc1W_ref[...], wfcW_ref[...],
                preferred_element_type=jnp.float32)
    c0 = jnp.dot(wfc1W_ref[...], wfcb_ref[...],
                 preferred_element_type=jnp.float32) + wfc1b_ref[...]
    ew_ref[...] = (dis_ref[...] * A[0:1, 0:1] + spec_ref[...] * A[0:1, 1:2]
                   + c0[0:1, 0:1])


def _lstm_step(msg_ref, ewq_ref, sel_ref, h_s, c_s,
               wihB_ref, whhB_ref, bB_ref, pi_ref, pf_ref, pg_ref, po_ref,
               P, GW, slot):
    t = pl.program_id(1)

    @pl.when(t == 0)
    def _():
        h_s[...] = jnp.zeros(h_s.shape, jnp.float32)
        c_s[...] = jnp.zeros(c_s.shape, jnp.float32)

    h = h_s[...]
    c = c_s[...]
    ewsel = jnp.dot(ewq_ref[...], sel_ref[0],
                    preferred_element_type=jnp.float32)
    xt = msg_ref[0]
    g = (jnp.dot(xt, wihB_ref[...], preferred_element_type=jnp.float32) * ewsel
         + jnp.dot(h, whhB_ref[...], preferred_element_type=jnp.float32)
         + bB_ref[...])
    th = jnp.tanh(g)
    lane = lax.broadcasted_iota(jnp.int32, g.shape, 1)
    is_g = (lane % GW) // slot == 2
    act = jnp.where(is_g, th, 0.5 * th + 0.5)
    i = jnp.dot(act, pi_ref[...], preferred_element_type=jnp.float32)
    f = jnp.dot(act, pf_ref[...], preferred_element_type=jnp.float32)
    gg = jnp.dot(act, pg_ref[...], preferred_element_type=jnp.float32)
    o = jnp.dot(act, po_ref[...], preferred_element_type=jnp.float32)
    c2 = f * c + i * gg
    h2 = o * jnp.tanh(c2)
    h_s[...] = h2
    c_s[...] = c2
    return h2


def _lstm1_kernel(msg_ref, ewq_ref, sel_ref, feat_ref,
                  wihB_ref, whhB_ref, bB_ref, pi_ref, pf_ref, pg_ref, po_ref,
                  WsB_ref, WnB_ref, cbB_ref, h1a_ref, h1b_ref, h_s, c_s):
    h = _lstm_step(msg_ref, ewq_ref, sel_ref, h_s, c_s,
                   wihB_ref, whhB_ref, bB_ref,
                   pi_ref, pf_ref, pg_ref, po_ref, 1250, 32, 8)

    @pl.when(pl.program_id(1) == K - 1)
    def _():
        out = (jnp.dot(feat_ref[...], WsB_ref[...], preferred_element_type=jnp.float32)
               + jnp.dot(h, WnB_ref[...], preferred_element_type=jnp.float32)
               + cbB_ref[...])
        out = _leaky(out)
        h1a_ref[...] = out[:, 0:128]
        h1b_ref[...] = out[:, 128:256]


def _lstm2_kernel(msg_ref, ewq_ref, sel_ref, h1_ref,
                  wihB_ref, whhB_ref, bB_ref, pi_ref, pf_ref, pg_ref, po_ref,
                  WsB_ref, WnB_ref, cbB_ref, scale_ref, shift_ref, h2_ref,
                  h_s, c_s):
    h = _lstm_step(msg_ref, ewq_ref, sel_ref, h_s, c_s,
                   wihB_ref, whhB_ref, bB_ref,
                   pi_ref, pf_ref, pg_ref, po_ref, 2500, 128, 32)

    @pl.when(pl.program_id(1) == K - 1)
    def _():
        out = (jnp.dot(h1_ref[...], WsB_ref[...], preferred_element_type=jnp.float32)
               + jnp.dot(h, WnB_ref[...], preferred_element_type=jnp.float32)
               + cbB_ref[...])
        h2_ref[...] = _leaky(out * scale_ref[...] + shift_ref[...])


def _head_kernel(h2_ref, gWrep_ref, f1W_ref, f1b_ref, f2W_ref, f2b_ref,
                 out_ref):
    h2 = h2_ref[...]
    l = jnp.dot(h2, gWrep_ref[...], preferred_element_type=jnp.float32)
    m = jnp.max(l)
    e = jnp.exp(l - m)
    s = jnp.sum(e, axis=0, keepdims=True)
    gate = e / s
    pooled = jnp.sum(gate[:, 0:20] * h2, axis=0, keepdims=True)
    o1 = _leaky(jnp.dot(pooled, f1W_ref[...].T,
                        preferred_element_type=jnp.float32) + f1b_ref[...])
    out_ref[...] = jnp.dot(o1, f2W_ref[...].T,
                           preferred_element_type=jnp.float32) + f2b_ref[...]


def _np_blockdiag(block, n):
    r, c = block.shape
    out = np.zeros((n * r, n * c), np.float32)
    for j in range(n):
        out[j * r:(j + 1) * r, j * c:(j + 1) * c] = block
    return out


def _jnp_blockdiag(block, n):
    r, c = block.shape
    out = jnp.zeros((n * r, n * c), jnp.float32)
    for j in range(n):
        out = out.at[j * r:(j + 1) * r, j * c:(j + 1) * c].set(block)
    return out


@functools.lru_cache()
def _gate_halver(npk, GW, slot):
    v = np.full((1, npk * GW), 0.5, np.float32)
    for j in range(npk):
        v[0, j * GW + 2 * slot:j * GW + 3 * slot] = 1.0
    return v


@functools.lru_cache()
def _sel_const(npk, GW):
    sel = np.zeros((K, npk * K, npk * GW), np.float32)
    for t in range(K):
        for j in range(npk):
            sel[t, j * K + t, j * GW:(j + 1) * GW] = 1.0
    return sel


@functools.lru_cache()
def _extract_const(npk, GW, slot, H):
    ps = []
    for kgate in range(4):
        pm = np.zeros((npk * GW, npk * H), np.float32)
        for j in range(npk):
            for l in range(H):
                pm[j * GW + kgate * slot + l, j * H + l] = 1.0
        ps.append(pm)
    return ps


def kernel(x, dis, spec, edge_src, params):
    p = params
    x2d = x.reshape(N, CN)
    dis2d = dis.reshape(E // 128, 128)
    spec2d = spec.reshape(E // 128, 128)

    feat_pad, ew2d = pl.pallas_call(
        _prep_kernel,
        out_shape=(jax.ShapeDtypeStruct((N, 16), jnp.float32),
                   jax.ShapeDtypeStruct((E // 128, 128), jnp.float32)),
    )(x2d, dis2d, spec2d,
      p['hfc_W'], p['hfc_b'].reshape(1, CN), p['attn_W'],
      p['wfc_W'], p['wfc_b'].reshape(100, 1), p['wfc1_W'],
      p['wfc1_b'].reshape(1, 1))

    info = plsc.get_sparse_core_info()
    nw = info.num_cores * info.num_subcores
    idx_t = edge_src.reshape(N, K).T.reshape(nw, E // nw)

    msg1 = _sc_gather(feat_pad, idx_t, 16)
    msg1p = msg1.reshape(K, N * 16 // 128, 128)

    wih1g = jnp.zeros((16, 32), jnp.float32)
    whh1g = jnp.zeros((8, 32), jnp.float32)
    b1g = jnp.zeros((1, 32), jnp.float32)
    bsum1 = p['l1_bih'] + p['l1_bhh']
    for gi in range(4):
        wih1g = wih1g.at[0:CN, gi * 8:gi * 8 + CN].set(
            p['l1_Wih'][gi * CN:(gi + 1) * CN, :].T)
        whh1g = whh1g.at[0:CN, gi * 8:gi * 8 + CN].set(
            p['l1_Whh'][gi * CN:(gi + 1) * CN, :].T)
        b1g = b1g.at[0, gi * 8:gi * 8 + CN].set(bsum1[gi * CN:(gi + 1) * CN])
    hv1 = jnp.asarray(_gate_halver(8, 32, 8))
    wihB1 = _jnp_blockdiag(wih1g, 8) * hv1
    whhB1 = _jnp_blockdiag(whh1g, 8) * hv1
    bB1 = jnp.tile(b1g, (1, 8)) * hv1
    sel1 = jnp.asarray(_sel_const(8, 32))
    p1 = [jnp.asarray(m) for m in _extract_const(8, 32, 8, 8)]
    ws1g = jnp.zeros((16, 32), jnp.float32).at[0:CN, :].set(p['c1_Ws'].T)
    wn1g = jnp.zeros((8, 32), jnp.float32).at[0:CN, :].set(p['c1_Wn'].T)
    WsB1 = _jnp_blockdiag(ws1g, 8)
    WnB1 = _jnp_blockdiag(wn1g, 8)
    cbB1 = jnp.tile(p['c1_b'].reshape(1, 32), (1, 8))
    ew8 = ew2d.reshape(N * K // 256, 256)

    def full(shape):
        return pl.BlockSpec(shape, lambda i, t: tuple(0 for _ in shape))

    h1a, h1b = pl.pallas_call(
        _lstm1_kernel,
        grid=(1, K),
        in_specs=[
            pl.BlockSpec((1, 1250, 128), lambda i, t: (t, i, 0)),
            pl.BlockSpec((1250, 256), lambda i, t: (i, 0)),
            pl.BlockSpec((1, 256, 256), lambda i, t: (t, 0, 0)),
            pl.BlockSpec((1250, 128), lambda i, t: (i, 0)),
            full((128, 256)), full((64, 256)), full((1, 256)),
            full((256, 64)), full((256, 64)), full((256, 64)), full((256, 64)),
            full((128, 256)), full((64, 256)), full((1, 256)),
        ],
        out_specs=[pl.BlockSpec((1250, 128), lambda i, t: (i, 0)),
                   pl.BlockSpec((1250, 128), lambda i, t: (i, 0))],
        out_shape=[jax.ShapeDtypeStruct((1250, 128), jnp.float32),
                   jax.ShapeDtypeStruct((1250, 128), jnp.float32)],
        scratch_shapes=[pltpu.VMEM((1250, 64), jnp.float32),
                        pltpu.VMEM((1250, 64), jnp.float32)],
    )(msg1p, ew8, sel1, feat_pad.reshape(1250, 128),
      wihB1, whhB1, bB1, *p1, WsB1, WnB1, cbB1)

    h1p = jnp.stack([h1a, h1b], axis=1).reshape(2500, 128)
    h1_table = h1p.reshape(N, 32)

    msg2 = _sc_gather(h1_table, idx_t, 32)
    msg2p = msg2.reshape(K, N * 32 // 128, 128)

    wih2g = jnp.concatenate(
        [p['l2_Wih'][gi * 32:(gi + 1) * 32, :].T for gi in range(4)], axis=1)
    whh2g = jnp.concatenate(
        [p['l2_Whh'][gi * 32:(gi + 1) * 32, :].T for gi in range(4)], axis=1)
    b2g = (p['l2_bih'] + p['l2_bhh']).reshape(1, 128)
    hv2 = jnp.asarray(_gate_halver(4, 128, 32))
    wihB2 = _jnp_blockdiag(wih2g, 4) * hv2
    whhB2 = _jnp_blockdiag(whh2g, 4) * hv2
    bB2 = jnp.tile(b2g, (1, 4)) * hv2
    sel2 = jnp.asarray(_sel_const(4, 128))
    p2 = [jnp.asarray(m) for m in _extract_const(4, 128, 32, 32)]
    WsB2 = _jnp_blockdiag(p['c2_Ws'].T, 4)
    WnB2 = _jnp_blockdiag(p['c2_Wn'].T, 4)
    cbB2 = jnp.tile(p['c2_b'].reshape(1, 20), (1, 4))
    scale20 = p['bn_g'] * lax.rsqrt(p['bn_rv'] + 1e-5)
    shift20 = p['bn_b'] - p['bn_rm'] * scale20
    scaleB = jnp.tile(scale20.reshape(1, 20), (1, 4))
    shiftB = jnp.tile(shift20.reshape(1, 20), (1, 4))

    h2p = pl.pallas_call(
        _lstm2_kernel,
        grid=(1, K),
        in_specs=[
            pl.BlockSpec((1, 2500, 128), lambda i, t: (t, i, 0)),
            pl.BlockSpec((2500, 128), lambda i, t: (i, 0)),
            pl.BlockSpec((1, 128, 512), lambda i, t: (t, 0, 0)),
            pl.BlockSpec((2500, 128), lambda i, t: (i, 0)),
            full((128, 512)), full((128, 512)), full((1, 512)),
            full((512, 128)), full((512, 128)), full((512, 128)), full((512, 128)),
            full((128, 80)), full((128, 80)), full((1, 80)),
            full((1, 80)), full((1, 80)),
        ],
        out_specs=pl.BlockSpec((2500, 80), lambda i, t: (i, 0)),
        out_shape=jax.ShapeDtypeStruct((2500, 80), jnp.float32),
        scratch_shapes=[pltpu.VMEM((2500, 128), jnp.float32),
                        pltpu.VMEM((2500, 128), jnp.float32)],
    )(msg2p, ew2d, sel2, h1p,
      wihB2, whhB2, bB2, *p2, WsB2, WnB2, cbB2, scaleB, shiftB)

    h2 = h2p.reshape(N, 20)

    out = pl.pallas_call(
        _head_kernel,
        out_shape=jax.ShapeDtypeStruct((1, 2), jnp.float32),
    )(h2, jnp.tile(p['gate_W'].T, (1, 128)),
      p['fc1_W'], p['fc1_b'].reshape(1, 10),
      p['fc2_W'], p['fc2_b'].reshape(1, 2))

    return out


# speedup vs baseline: 5.5791x; 1.2052x over previous
import functools

import numpy as np
import jax
import jax.numpy as jnp
from jax import lax
from jax.experimental import pallas as pl
from jax.experimental.pallas import tpu as pltpu
from jax.experimental.pallas import tpu_sc as plsc

N = 10000
K = 32
CN = 6
E = N * K

_LEAK = 0.01


def _leaky(x):
    return jnp.where(x >= 0, x, _LEAK * x)


def _sc_gather(table, idx_2d, D):
    info = plsc.get_sparse_core_info()
    nc, ns = info.num_cores, info.num_subcores
    nw = nc * ns
    per_w = idx_2d.shape[1]
    n_full = per_w // 128
    tail = per_w - n_full * 128
    nbuf = 6
    n_super = n_full // nbuf
    S = nbuf * 128
    mesh = plsc.VectorSubcoreMesh(core_axis_name="c", subcore_axis_name="s")

    @functools.partial(
        pl.kernel,
        mesh=mesh,
        compiler_params=pltpu.CompilerParams(use_tc_tiling_on_sc=False),
        out_type=jax.ShapeDtypeStruct((nw * per_w, D), jnp.float32),
        scratch_types=[
            pltpu.VMEM((per_w,), jnp.int32),
            pltpu.VMEM((S, D), jnp.float32),
            pltpu.SemaphoreType.DMA,
        ],
    )
    def k(table_hbm, idx_hbm, out_hbm, idx_v, buf, sem):
        wid = lax.axis_index("s") * nc + lax.axis_index("c")
        base = wid * per_w
        pltpu.sync_copy(idx_hbm.at[wid], idx_v)

        def body(sg, carry):
            cps = []
            for b in range(nbuf):
                cps.append(pltpu.async_copy(
                    table_hbm.at[idx_v.at[pl.ds((sg * nbuf + b) * 128, 128)]],
                    buf.at[pl.ds(b * 128, 128)], sem))
            for cp in cps:
                cp.wait()
            pltpu.sync_copy(buf, out_hbm.at[pl.ds(base + sg * S, S)])
            return carry

        lax.fori_loop(0, n_super, body, 0)
        if tail:
            pltpu.async_copy(
                table_hbm.at[idx_v.at[pl.ds(n_full * 128, tail)]],
                buf.at[pl.ds(0, tail)], sem).wait()
            pltpu.sync_copy(buf.at[pl.ds(0, tail)],
                            out_hbm.at[pl.ds(base + n_full * 128, tail)])

    return k(table, idx_2d)


def _prep_kernel(x_ref, dis_ref, spec_ref, hfcW_ref, hfcb_ref, attnW_ref,
                 wfcW_ref, wfcb_ref, wfc1W_ref, wfc1b_ref,
                 feat_ref, ew_ref):
    xv = x_ref[...]
    z = jnp.dot(xv, hfcW_ref[...].T,
                preferred_element_type=jnp.float32) + hfcb_ref[...]
    w1 = attnW_ref[:, 0:CN]
    s = jnp.sum(attnW_ref[:, CN:2 * CN])
    c = jnp.sum(z * w1, axis=1, keepdims=True)
    a = _leaky(c + s * z)
    m = jnp.max(a, axis=1, keepdims=True)
    e = jnp.exp(a - m)
    alpha = e / jnp.sum(e, axis=1, keepdims=True)
    feat = alpha * z
    feat_ref[...] = jnp.concatenate(
        [feat, jnp.zeros((feat.shape[0], 16 - CN), jnp.float32)], axis=1)

    A = jnp.dot(wfc1W_ref[...], wfcW_ref[...],
                preferred_element_type=jnp.float32)
    c0 = jnp.dot(wfc1W_ref[...], wfcb_ref[...],
                 preferred_element_type=jnp.float32) + wfc1b_ref[...]
    ew_ref[...] = (dis_ref[...] * A[0:1, 0:1] + spec_ref[...] * A[0:1, 1:2]
                   + c0[0:1, 0:1])


def _dot(a, b):
    return jnp.dot(a, b, preferred_element_type=jnp.float32)


_UNROLL = 2


def _lstm_steps(msg_ref, ewq_ref, selx_ref, h_s, c_s, wih, whh, bias):
    t = pl.program_id(1)

    @pl.when(t == 0)
    def _():
        h_s[...] = jnp.zeros(h_s.shape, jnp.float32)
        c_s[...] = jnp.zeros(c_s.shape, jnp.float32)

    h = h_s[...]
    c = c_s[...]
    for u in range(_UNROLL):
        ewx = _dot(ewq_ref[...], selx_ref[u])
        xts = msg_ref[u] * ewx
        g_i = _dot(xts, wih[0][...]) + _dot(h, whh[0][...]) + bias[0][...]
        g_f = _dot(xts, wih[1][...]) + _dot(h, whh[1][...]) + bias[1][...]
        g_g = _dot(xts, wih[2][...]) + _dot(h, whh[2][...]) + bias[2][...]
        g_o = _dot(xts, wih[3][...]) + _dot(h, whh[3][...]) + bias[3][...]
        i = 0.5 * jnp.tanh(g_i) + 0.5
        f = 0.5 * jnp.tanh(g_f) + 0.5
        gg = jnp.tanh(g_g)
        o = 0.5 * jnp.tanh(g_o) + 0.5
        c = f * c + i * gg
        h = o * jnp.tanh(c)
    h_s[...] = h
    c_s[...] = c
    return h


def _lstm1_kernel(msg_ref, ewq_ref, selx_ref, feat_ref,
                  wih0, wih1, wih2, wih3, whh0, whh1, whh2, whh3,
                  b0, b1, b2, b3,
                  WsB_ref, WnB_ref, cbB_ref, h1a_ref, h1b_ref, h_s, c_s):
    h = _lstm_steps(msg_ref, ewq_ref, selx_ref, h_s, c_s,
                    (wih0, wih1, wih2, wih3), (whh0, whh1, whh2, whh3),
                    (b0, b1, b2, b3))

    @pl.when(pl.program_id(1) == K // _UNROLL - 1)
    def _():
        out = (_dot(feat_ref[...], WsB_ref[...]) + _dot(h, WnB_ref[...])
               + cbB_ref[...])
        out = _leaky(out)
        h1a_ref[...] = out[:, 0:128]
        h1b_ref[...] = out[:, 128:256]


def _lstm2_kernel(msg_ref, ewq_ref, selx_ref, h1_ref,
                  wih0, wih1, wih2, wih3, whh0, whh1, whh2, whh3,
                  b0, b1, b2, b3,
                  WsB_ref, WnB_ref, cbB_ref, scale_ref, shift_ref,
                  gw4_ref, rep_ref, fold_ref, f1W_ref, f1b_ref,
                  f2W_ref, f2b_ref, out_ref, h_s, c_s):
    h = _lstm_steps(msg_ref, ewq_ref, selx_ref, h_s, c_s,
                    (wih0, wih1, wih2, wih3), (whh0, whh1, whh2, whh3),
                    (b0, b1, b2, b3))

    @pl.when(pl.program_id(1) == K // _UNROLL - 1)
    def _():
        h2p = (_dot(h1_ref[...], WsB_ref[...]) + _dot(h, WnB_ref[...])
               + cbB_ref[...])
        h2p = _leaky(h2p * scale_ref[...] + shift_ref[...])
        l4 = _dot(h2p, gw4_ref[...])
        m = jnp.max(l4)
        e4 = jnp.exp(l4 - m)
        s = jnp.sum(e4)
        erep = _dot(e4, rep_ref[...])
        p80 = jnp.sum(erep * h2p, axis=0, keepdims=True)
        pooled = _dot(p80, fold_ref[...]) / s
        o1 = _leaky(_dot(pooled, f1W_ref[...]) + f1b_ref[...])
        out_ref[...] = _dot(o1, f2W_ref[...]) + f2b_ref[...]


def _np_blockdiag(block, n):
    r, c = block.shape
    out = np.zeros((n * r, n * c), np.float32)
    for j in range(n):
        out[j * r:(j + 1) * r, j * c:(j + 1) * c] = block
    return out


def _jnp_blockdiag(block, n):
    r, c = block.shape
    out = jnp.zeros((n * r, n * c), jnp.float32)
    for j in range(n):
        out = out.at[j * r:(j + 1) * r, j * c:(j + 1) * c].set(block)
    return out


@functools.lru_cache()
def _sel_const(npk, GW):
    sel = np.zeros((K, npk * K, npk * GW), np.float32)
    for t in range(K):
        for j in range(npk):
            sel[t, j * K + t, j * GW:(j + 1) * GW] = 1.0
    return sel


def kernel(x, dis, spec, edge_src, params):
    p = params
    x2d = x.reshape(N, CN)
    dis2d = dis.reshape(E // 128, 128)
    spec2d = spec.reshape(E // 128, 128)

    feat_pad, ew2d = pl.pallas_call(
        _prep_kernel,
        out_shape=(jax.ShapeDtypeStruct((N, 16), jnp.float32),
                   jax.ShapeDtypeStruct((E // 128, 128), jnp.float32)),
    )(x2d, dis2d, spec2d,
      p['hfc_W'], p['hfc_b'].reshape(1, CN), p['attn_W'],
      p['wfc_W'], p['wfc_b'].reshape(100, 1), p['wfc1_W'],
      p['wfc1_b'].reshape(1, 1))

    info = plsc.get_sparse_core_info()
    nw = info.num_cores * info.num_subcores
    idx_t = edge_src.reshape(N, K).T.reshape(nw, E // nw)

    msg1 = _sc_gather(feat_pad, idx_t, 16)
    msg1p = msg1.reshape(K, N * 16 // 128, 128)

    bsum1 = p['l1_bih'] + p['l1_bhh']
    wihB1, whhB1, bB1 = [], [], []
    for gi in range(4):
        sc = 1.0 if gi == 2 else 0.5
        wg = jnp.pad(p['l1_Wih'][gi * CN:(gi + 1) * CN, :].T * sc,
                     ((0, 10), (0, 2)))
        hg = jnp.pad(p['l1_Whh'][gi * CN:(gi + 1) * CN, :].T * sc,
                     ((0, 2), (0, 2)))
        bg = jnp.pad(bsum1[gi * CN:(gi + 1) * CN] * sc, (0, 2)).reshape(1, 8)
        wihB1.append(_jnp_blockdiag(wg, 8))
        whhB1.append(_jnp_blockdiag(hg, 8))
        bB1.append(jnp.tile(bg, (1, 8)))
    selx1 = jnp.asarray(_sel_const(8, 16))
    ws1g = jnp.zeros((16, 32), jnp.float32).at[0:CN, :].set(p['c1_Ws'].T)
    wn1g = jnp.zeros((8, 32), jnp.float32).at[0:CN, :].set(p['c1_Wn'].T)
    WsB1 = _jnp_blockdiag(ws1g, 8)
    WnB1 = _jnp_blockdiag(wn1g, 8)
    cbB1 = jnp.tile(p['c1_b'].reshape(1, 32), (1, 8))
    ew8 = ew2d.reshape(N * K // 256, 256)

    def full(shape):
        return pl.BlockSpec(shape, lambda i, t: tuple(0 for _ in shape))

    h1a, h1b = pl.pallas_call(
        _lstm1_kernel,
        grid=(1, K // _UNROLL),
        in_specs=[
            pl.BlockSpec((_UNROLL, 1250, 128), lambda i, t: (t, i, 0)),
            pl.BlockSpec((1250, 256), lambda i, t: (i, 0)),
            pl.BlockSpec((_UNROLL, 256, 128), lambda i, t: (t, 0, 0)),
            pl.BlockSpec((1250, 128), lambda i, t: (i, 0)),
            *[full((128, 64))] * 4, *[full((64, 64))] * 4, *[full((1, 64))] * 4,
            full((128, 256)), full((64, 256)), full((1, 256)),
        ],
        out_specs=[pl.BlockSpec((1250, 128), lambda i, t: (i, 0)),
                   pl.BlockSpec((1250, 128), lambda i, t: (i, 0))],
        out_shape=[jax.ShapeDtypeStruct((1250, 128), jnp.float32),
                   jax.ShapeDtypeStruct((1250, 128), jnp.float32)],
        scratch_shapes=[pltpu.VMEM((1250, 64), jnp.float32),
                        pltpu.VMEM((1250, 64), jnp.float32)],
    )(msg1p, ew8, selx1, feat_pad.reshape(1250, 128),
      *wihB1, *whhB1, *bB1, WsB1, WnB1, cbB1)

    h1p = jnp.stack([h1a, h1b], axis=1).reshape(2500, 128)
    h1_table = h1p.reshape(N, 32)

    msg2 = _sc_gather(h1_table, idx_t, 32)
    msg2p = msg2.reshape(K, N * 32 // 128, 128)

    bsum2 = p['l2_bih'] + p['l2_bhh']
    wihB2, whhB2, bB2 = [], [], []
    for gi in range(4):
        sc = 1.0 if gi == 2 else 0.5
        wihB2.append(_jnp_blockdiag(p['l2_Wih'][gi * 32:(gi + 1) * 32, :].T * sc, 4))
        whhB2.append(_jnp_blockdiag(p['l2_Whh'][gi * 32:(gi + 1) * 32, :].T * sc, 4))
        bB2.append(jnp.tile(bsum2[gi * 32:(gi + 1) * 32].reshape(1, 32) * sc, (1, 4)))
    selx2 = jnp.asarray(_sel_const(4, 32))
    WsB2 = _jnp_blockdiag(p['c2_Ws'].T, 4)
    WnB2 = _jnp_blockdiag(p['c2_Wn'].T, 4)
    cbB2 = jnp.tile(p['c2_b'].reshape(1, 20), (1, 4))
    scale20 = p['bn_g'] * lax.rsqrt(p['bn_rv'] + 1e-5)
    shift20 = p['bn_b'] - p['bn_rm'] * scale20
    scaleB = jnp.tile(scale20.reshape(1, 20), (1, 4))
    shiftB = jnp.tile(shift20.reshape(1, 20), (1, 4))
    gw4 = _jnp_blockdiag(p['gate_W'].T, 4)
    rep = jnp.asarray(_np_blockdiag(np.ones((1, 20), np.float32), 4))
    fold = jnp.asarray(np.tile(np.eye(20, dtype=np.float32), (4, 1)))

    out = pl.pallas_call(
        _lstm2_kernel,
        grid=(1, K // _UNROLL),
        in_specs=[
            pl.BlockSpec((_UNROLL, 2500, 128), lambda i, t: (t, i, 0)),
            pl.BlockSpec((2500, 128), lambda i, t: (i, 0)),
            pl.BlockSpec((_UNROLL, 128, 128), lambda i, t: (t, 0, 0)),
            pl.BlockSpec((2500, 128), lambda i, t: (i, 0)),
            *[full((128, 128))] * 8, *[full((1, 128))] * 4,
            full((128, 80)), full((128, 80)), full((1, 80)),
            full((1, 80)), full((1, 80)),
            full((80, 4)), full((4, 80)), full((80, 20)),
            full((20, 10)), full((1, 10)), full((10, 2)), full((1, 2)),
        ],
        out_specs=pl.BlockSpec((1, 2), lambda i, t: (0, 0)),
        out_shape=jax.ShapeDtypeStruct((1, 2), jnp.float32),
        scratch_shapes=[pltpu.VMEM((2500, 128), jnp.float32),
                        pltpu.VMEM((2500, 128), jnp.float32)],
    )(msg2p, ew2d, selx2, h1p,
      *wihB2, *whhB2, *bB2, WsB2, WnB2, cbB2, scaleB, shiftB,
      gw4, rep, fold, p['fc1_W'].T, p['fc1_b'].reshape(1, 10),
      p['fc2_W'].T, p['fc2_b'].reshape(1, 2))

    return out


# speedup vs baseline: 5.6234x; 1.0079x over previous
import functools

import numpy as np
import jax
import jax.numpy as jnp
from jax import lax
from jax.experimental import pallas as pl
from jax.experimental.pallas import tpu as pltpu
from jax.experimental.pallas import tpu_sc as plsc

N = 10000
K = 32
CN = 6
E = N * K

_LEAK = 0.01


def _leaky(x):
    return jnp.where(x >= 0, x, _LEAK * x)


def _sc_gather(table, idx_2d, D):
    info = plsc.get_sparse_core_info()
    nc, ns = info.num_cores, info.num_subcores
    nw = nc * ns
    per_w = idx_2d.shape[1]
    n_full = per_w // 128
    tail = per_w - n_full * 128
    nbuf = 13
    n_super = n_full // nbuf
    S = nbuf * 128
    mesh = plsc.VectorSubcoreMesh(core_axis_name="c", subcore_axis_name="s")

    @functools.partial(
        pl.kernel,
        mesh=mesh,
        compiler_params=pltpu.CompilerParams(use_tc_tiling_on_sc=False),
        out_type=jax.ShapeDtypeStruct((nw * per_w, D), jnp.float32),
        scratch_types=[
            pltpu.VMEM((per_w,), jnp.int32),
            pltpu.VMEM((S, D), jnp.float32),
            pltpu.SemaphoreType.DMA,
        ],
    )
    def k(table_hbm, idx_hbm, out_hbm, idx_v, buf, sem):
        wid = lax.axis_index("s") * nc + lax.axis_index("c")
        base = wid * per_w
        pltpu.sync_copy(idx_hbm.at[wid], idx_v)

        def body(sg, carry):
            cps = []
            for b in range(nbuf):
                cps.append(pltpu.async_copy(
                    table_hbm.at[idx_v.at[pl.ds((sg * nbuf + b) * 128, 128)]],
                    buf.at[pl.ds(b * 128, 128)], sem))
            for cp in cps:
                cp.wait()
            pltpu.sync_copy(buf, out_hbm.at[pl.ds(base + sg * S, S)])
            return carry

        lax.fori_loop(0, n_super, body, 0)
        if tail:
            pltpu.async_copy(
                table_hbm.at[idx_v.at[pl.ds(n_full * 128, tail)]],
                buf.at[pl.ds(0, tail)], sem).wait()
            pltpu.sync_copy(buf.at[pl.ds(0, tail)],
                            out_hbm.at[pl.ds(base + n_full * 128, tail)])

    return k(table, idx_2d)


def _prep_kernel(x_ref, dis_ref, spec_ref, hfcW_ref, hfcb_ref, attnW_ref,
                 wfcW_ref, wfcb_ref, wfc1W_ref, wfc1b_ref,
                 feat_ref, ew_ref):
    xv = x_ref[...]
    z = jnp.dot(xv, hfcW_ref[...].T,
                preferred_element_type=jnp.float32) + hfcb_ref[...]
    w1 = attnW_ref[:, 0:CN]
    s = jnp.sum(attnW_ref[:, CN:2 * CN])
    c = jnp.sum(z * w1, axis=1, keepdims=True)
    a = _leaky(c + s * z)
    m = jnp.max(a, axis=1, keepdims=True)
    e = jnp.exp(a - m)
    alpha = e / jnp.sum(e, axis=1, keepdims=True)
    feat = alpha * z
    feat_ref[...] = jnp.concatenate(
        [feat, jnp.zeros((feat.shape[0], 16 - CN), jnp.float32)], axis=1)

    A = jnp.dot(wfc1W_ref[...], wfcW_ref[...],
                preferred_element_type=jnp.float32)
    c0 = jnp.dot(wfc1W_ref[...], wfcb_ref[...],
                 preferred_element_type=jnp.float32) + wfc1b_ref[...]
    ew_ref[...] = (dis_ref[...] * A[0:1, 0:1] + spec_ref[...] * A[0:1, 1:2]
                   + c0[0:1, 0:1])


def _dot(a, b):
    return jnp.dot(a, b, preferred_element_type=jnp.float32)


_UNROLL = 2


def _lstm_steps(msg_ref, ewq_ref, selx_ref, h_s, c_s, wih, whh, bias):
    t = pl.program_id(1)

    @pl.when(t == 0)
    def _():
        h_s[...] = jnp.zeros(h_s.shape, jnp.float32)
        c_s[...] = jnp.zeros(c_s.shape, jnp.float32)

    h = h_s[...]
    c = c_s[...]
    for u in range(_UNROLL):
        ewx = _dot(ewq_ref[...], selx_ref[u])
        xts = msg_ref[u] * ewx
        g_i = _dot(xts, wih[0][...]) + _dot(h, whh[0][...]) + bias[0][...]
        g_f = _dot(xts, wih[1][...]) + _dot(h, whh[1][...]) + bias[1][...]
        g_g = _dot(xts, wih[2][...]) + _dot(h, whh[2][...]) + bias[2][...]
        g_o = _dot(xts, wih[3][...]) + _dot(h, whh[3][...]) + bias[3][...]
        i = 0.5 * jnp.tanh(g_i) + 0.5
        f = 0.5 * jnp.tanh(g_f) + 0.5
        gg = jnp.tanh(g_g)
        o = 0.5 * jnp.tanh(g_o) + 0.5
        c = f * c + i * gg
        h = o * jnp.tanh(c)
    h_s[...] = h
    c_s[...] = c
    return h


def _lstm1_kernel(msg_ref, ewq_ref, selx_ref, feat_ref,
                  wih0, wih1, wih2, wih3, whh0, whh1, whh2, whh3,
                  b0, b1, b2, b3,
                  WsB_ref, WnB_ref, cbB_ref, h1a_ref, h1b_ref, h_s, c_s):
    h = _lstm_steps(msg_ref, ewq_ref, selx_ref, h_s, c_s,
                    (wih0, wih1, wih2, wih3), (whh0, whh1, whh2, whh3),
                    (b0, b1, b2, b3))

    @pl.when(pl.program_id(1) == K // _UNROLL - 1)
    def _():
        out = (_dot(feat_ref[...], WsB_ref[...]) + _dot(h, WnB_ref[...])
               + cbB_ref[...])
        out = _leaky(out)
        h1a_ref[...] = out[:, 0:128]
        h1b_ref[...] = out[:, 128:256]


def _lstm2_kernel(msg_ref, ewq_ref, selx_ref, h1_ref,
                  wih0, wih1, wih2, wih3, whh0, whh1, whh2, whh3,
                  b0, b1, b2, b3,
                  WsB_ref, WnB_ref, cbB_ref, scale_ref, shift_ref,
                  gw4_ref, rep_ref, fold_ref, f1W_ref, f1b_ref,
                  f2W_ref, f2b_ref, out_ref, h_s, c_s):
    h = _lstm_steps(msg_ref, ewq_ref, selx_ref, h_s, c_s,
                    (wih0, wih1, wih2, wih3), (whh0, whh1, whh2, whh3),
                    (b0, b1, b2, b3))

    @pl.when(pl.program_id(1) == K // _UNROLL - 1)
    def _():
        h2p = (_dot(h1_ref[...], WsB_ref[...]) + _dot(h, WnB_ref[...])
               + cbB_ref[...])
        h2p = _leaky(h2p * scale_ref[...] + shift_ref[...])
        l4 = _dot(h2p, gw4_ref[...])
        m = jnp.max(l4)
        e4 = jnp.exp(l4 - m)
        s = jnp.sum(e4)
        erep = _dot(e4, rep_ref[...])
        p80 = jnp.sum(erep * h2p, axis=0, keepdims=True)
        pooled = _dot(p80, fold_ref[...]) / s
        o1 = _leaky(_dot(pooled, f1W_ref[...]) + f1b_ref[...])
        out_ref[...] = _dot(o1, f2W_ref[...]) + f2b_ref[...]


def _np_blockdiag(block, n):
    r, c = block.shape
    out = np.zeros((n * r, n * c), np.float32)
    for j in range(n):
        out[j * r:(j + 1) * r, j * c:(j + 1) * c] = block
    return out


def _jnp_blockdiag(block, n):
    r, c = block.shape
    out = jnp.zeros((n * r, n * c), jnp.float32)
    for j in range(n):
        out = out.at[j * r:(j + 1) * r, j * c:(j + 1) * c].set(block)
    return out


@functools.lru_cache()
def _sel_const(npk, GW):
    sel = np.zeros((K, npk * K, npk * GW), np.float32)
    for t in range(K):
        for j in range(npk):
            sel[t, j * K + t, j * GW:(j + 1) * GW] = 1.0
    return sel


def kernel(x, dis, spec, edge_src, params):
    p = params
    x2d = x.reshape(N, CN)
    dis2d = dis.reshape(E // 128, 128)
    spec2d = spec.reshape(E // 128, 128)

    feat_pad, ew2d = pl.pallas_call(
        _prep_kernel,
        out_shape=(jax.ShapeDtypeStruct((N, 16), jnp.float32),
                   jax.ShapeDtypeStruct((E // 128, 128), jnp.float32)),
    )(x2d, dis2d, spec2d,
      p['hfc_W'], p['hfc_b'].reshape(1, CN), p['attn_W'],
      p['wfc_W'], p['wfc_b'].reshape(100, 1), p['wfc1_W'],
      p['wfc1_b'].reshape(1, 1))

    info = plsc.get_sparse_core_info()
    nw = info.num_cores * info.num_subcores
    idx_t = edge_src.reshape(N, K).T.reshape(nw, E // nw)

    msg1 = _sc_gather(feat_pad, idx_t, 16)
    msg1p = msg1.reshape(K, N * 16 // 128, 128)

    bsum1 = p['l1_bih'] + p['l1_bhh']
    wihB1, whhB1, bB1 = [], [], []
    for gi in range(4):
        sc = 1.0 if gi == 2 else 0.5
        wg = jnp.pad(p['l1_Wih'][gi * CN:(gi + 1) * CN, :].T * sc,
                     ((0, 10), (0, 2)))
        hg = jnp.pad(p['l1_Whh'][gi * CN:(gi + 1) * CN, :].T * sc,
                     ((0, 2), (0, 2)))
        bg = jnp.pad(bsum1[gi * CN:(gi + 1) * CN] * sc, (0, 2)).reshape(1, 8)
        wihB1.append(_jnp_blockdiag(wg, 8))
        whhB1.append(_jnp_blockdiag(hg, 8))
        bB1.append(jnp.tile(bg, (1, 8)))
    selx1 = jnp.asarray(_sel_const(8, 16))
    ws1g = jnp.zeros((16, 32), jnp.float32).at[0:CN, :].set(p['c1_Ws'].T)
    wn1g = jnp.zeros((8, 32), jnp.float32).at[0:CN, :].set(p['c1_Wn'].T)
    WsB1 = _jnp_blockdiag(ws1g, 8)
    WnB1 = _jnp_blockdiag(wn1g, 8)
    cbB1 = jnp.tile(p['c1_b'].reshape(1, 32), (1, 8))
    ew8 = ew2d.reshape(N * K // 256, 256)

    def full(shape):
        return pl.BlockSpec(shape, lambda i, t: tuple(0 for _ in shape))

    h1a, h1b = pl.pallas_call(
        _lstm1_kernel,
        grid=(1, K // _UNROLL),
        in_specs=[
            pl.BlockSpec((_UNROLL, 1250, 128), lambda i, t: (t, i, 0)),
            pl.BlockSpec((1250, 256), lambda i, t: (i, 0)),
            pl.BlockSpec((_UNROLL, 256, 128), lambda i, t: (t, 0, 0)),
            pl.BlockSpec((1250, 128), lambda i, t: (i, 0)),
            *[full((128, 64))] * 4, *[full((64, 64))] * 4, *[full((1, 64))] * 4,
            full((128, 256)), full((64, 256)), full((1, 256)),
        ],
        out_specs=[pl.BlockSpec((1250, 128), lambda i, t: (i, 0)),
                   pl.BlockSpec((1250, 128), lambda i, t: (i, 0))],
        out_shape=[jax.ShapeDtypeStruct((1250, 128), jnp.float32),
                   jax.ShapeDtypeStruct((1250, 128), jnp.float32)],
        scratch_shapes=[pltpu.VMEM((1250, 64), jnp.float32),
                        pltpu.VMEM((1250, 64), jnp.float32)],
    )(msg1p, ew8, selx1, feat_pad.reshape(1250, 128),
      *wihB1, *whhB1, *bB1, WsB1, WnB1, cbB1)

    h1p = jnp.stack([h1a, h1b], axis=1).reshape(2500, 128)
    h1_table = h1p.reshape(N, 32)

    msg2 = _sc_gather(h1_table, idx_t, 32)
    msg2p = msg2.reshape(K, N * 32 // 128, 128)

    bsum2 = p['l2_bih'] + p['l2_bhh']
    wihB2, whhB2, bB2 = [], [], []
    for gi in range(4):
        sc = 1.0 if gi == 2 else 0.5
        wihB2.append(_jnp_blockdiag(p['l2_Wih'][gi * 32:(gi + 1) * 32, :].T * sc, 4))
        whhB2.append(_jnp_blockdiag(p['l2_Whh'][gi * 32:(gi + 1) * 32, :].T * sc, 4))
        bB2.append(jnp.tile(bsum2[gi * 32:(gi + 1) * 32].reshape(1, 32) * sc, (1, 4)))
    selx2 = jnp.asarray(_sel_const(4, 32))
    WsB2 = _jnp_blockdiag(p['c2_Ws'].T, 4)
    WnB2 = _jnp_blockdiag(p['c2_Wn'].T, 4)
    cbB2 = jnp.tile(p['c2_b'].reshape(1, 20), (1, 4))
    scale20 = p['bn_g'] * lax.rsqrt(p['bn_rv'] + 1e-5)
    shift20 = p['bn_b'] - p['bn_rm'] * scale20
    scaleB = jnp.tile(scale20.reshape(1, 20), (1, 4))
    shiftB = jnp.tile(shift20.reshape(1, 20), (1, 4))
    gw4 = _jnp_blockdiag(p['gate_W'].T, 4)
    rep = jnp.asarray(_np_blockdiag(np.ones((1, 20), np.float32), 4))
    fold = jnp.asarray(np.tile(np.eye(20, dtype=np.float32), (4, 1)))

    out = pl.pallas_call(
        _lstm2_kernel,
        grid=(1, K // _UNROLL),
        in_specs=[
            pl.BlockSpec((_UNROLL, 2500, 128), lambda i, t: (t, i, 0)),
            pl.BlockSpec((2500, 128), lambda i, t: (i, 0)),
            pl.BlockSpec((_UNROLL, 128, 128), lambda i, t: (t, 0, 0)),
            pl.BlockSpec((2500, 128), lambda i, t: (i, 0)),
            *[full((128, 128))] * 8, *[full((1, 128))] * 4,
            full((128, 80)), full((128, 80)), full((1, 80)),
            full((1, 80)), full((1, 80)),
            full((80, 4)), full((4, 80)), full((80, 20)),
            full((20, 10)), full((1, 10)), full((10, 2)), full((1, 2)),
        ],
        out_specs=pl.BlockSpec((1, 2), lambda i, t: (0, 0)),
        out_shape=jax.ShapeDtypeStruct((1, 2), jnp.float32),
        scratch_shapes=[pltpu.VMEM((2500, 128), jnp.float32),
                        pltpu.VMEM((2500, 128), jnp.float32)],
    )(msg2p, ew2d, selx2, h1p,
      *wihB2, *whhB2, *bB2, WsB2, WnB2, cbB2, scaleB, shiftB,
      gw4, rep, fold, p['fc1_W'].T, p['fc1_b'].reshape(1, 10),
      p['fc2_W'].T, p['fc2_b'].reshape(1, 2))

    return out
